# baseline (speedup 1.0000x reference)
import functools

import jax
import jax.numpy as jnp
from jax import lax
from jax.experimental import pallas as pl
from jax.experimental.pallas import tpu as pltpu
from jax.experimental.pallas import tpu_sc as plsc

N = 10000
NPAD = 10240
E = 320000
F = 128
NG = 64
RB = 512
NRB = NPAD // RB
JC = 1024
NJC = NPAD // JC

NC = 2
NS = 16
NW = NC * NS
CH = 128
NCHUNK = 80
HC = NCHUNK // 2
NBUF = 2
EPW = NCHUNK * CH
EPAD = NW * EPW
ROWS_PER_SUB = NPAD // NS


def _sc_scatter_body(table_hbm, srcidx_hbm, dstidx_hbm, zero_hbm, out_hbm,
                     sidx_v, didx_v, rows0, rows1, acc_sh, gsem):
    rows = (rows0, rows1)
    cid = lax.axis_index("c")
    sid = lax.axis_index("s")
    w = cid * NS + sid
    r0 = sid * ROWS_PER_SUB
    pltpu.sync_copy(zero_hbm.at[pl.ds(r0, ROWS_PER_SUB)],
                    acc_sh.at[pl.ds(r0, ROWS_PER_SUB)])
    plsc.subcore_barrier()

    def _wait_gather(b):
        pltpu.make_async_copy(table_hbm.at[sidx_v.at[0]], rows[b],
                              gsem.at[b]).wait()

    for h in range(2):
        pltpu.sync_copy(srcidx_hbm.at[w, pl.ds(h * HC, HC)], sidx_v)
        pltpu.sync_copy(dstidx_hbm.at[w, pl.ds(h * HC, HC)], didx_v)
        pltpu.async_copy(table_hbm.at[sidx_v.at[0]], rows[0], gsem.at[0])

        def body(p, carry):
            j0 = 2 * p
            pltpu.async_copy(table_hbm.at[sidx_v.at[j0 + 1]], rows[1],
                             gsem.at[1])
            _wait_gather(0)
            pltpu.sync_copy(rows[0], acc_sh.at[didx_v.at[j0]], add=True)
            nxt = lax.rem(j0 + 2, HC)
            pltpu.async_copy(table_hbm.at[sidx_v.at[nxt]], rows[0],
                             gsem.at[0])
            _wait_gather(1)
            pltpu.sync_copy(rows[1], acc_sh.at[didx_v.at[j0 + 1]], add=True)
            return carry

        lax.fori_loop(0, HC // 2, body, 0, unroll=False)
        _wait_gather(0)
    plsc.subcore_barrier()
    pltpu.sync_copy(acc_sh.at[pl.ds(r0, ROWS_PER_SUB)],
                    out_hbm.at[cid, pl.ds(r0, ROWS_PER_SUB)])


def _make_sc_scatter(interpret=False):
    return pl.kernel(
        _sc_scatter_body,
        out_type=jax.ShapeDtypeStruct((NC, NPAD, F), jnp.float32),
        mesh=plsc.VectorSubcoreMesh(core_axis_name="c", subcore_axis_name="s",
                                    num_cores=NC, num_subcores=NS),
        scratch_types=[
            pltpu.VMEM((HC, CH), jnp.int32),
            pltpu.VMEM((HC, CH), jnp.int32),
            pltpu.VMEM((CH, F), jnp.float32),
            pltpu.VMEM((CH, F), jnp.float32),
            pltpu.VMEM_SHARED((NPAD, F), jnp.float32),
            pltpu.SemaphoreType.DMA((NBUF,)),
        ],
        interpret=interpret,
    )


_sc_scatter_cache = []


def _sc_scatter(*args):
    if not _sc_scatter_cache:
        _sc_scatter_cache.append(_make_sc_scatter())
    return _sc_scatter_cache[0](*args)


def _root_body(x_ref, wq_ref, b_ref, o_ref):
    o_ref[...] = jnp.dot(x_ref[...], wq_ref[...],
                         preferred_element_type=jnp.float32) + b_ref[...]


def _conv_body(xr_ref, pa_ref, pb_ref, wr_ref, pw_ref,
               xk_ref, s_ref, *, gated, gate_ref=None):
    agg = pa_ref[0] + pb_ref[0]
    if gated:
        agg = agg * gate_ref[...]
    h = jnp.dot(agg, wr_ref[...], preferred_element_type=jnp.float32)
    h = jnp.maximum(h + xr_ref[...], 0.0)
    w = pw_ref[...]
    nrm = jnp.sqrt(jnp.sum(w * w))
    s = jnp.tanh(jnp.dot(h, w, preferred_element_type=jnp.float32) / nrm)
    s_ref[...] = s
    xk_ref[...] = h * s


def _conv1_body(xr_ref, pa_ref, pb_ref, wr_ref, pw_ref, xk_ref, s_ref):
    _conv_body(xr_ref, pa_ref, pb_ref, wr_ref, pw_ref, xk_ref, s_ref,
               gated=False)


def _conv2_body(xr_ref, pa_ref, pb_ref, gate_ref, wr_ref, pw_ref,
                xk_ref, s_ref):
    _conv_body(xr_ref, pa_ref, pb_ref, wr_ref, pw_ref, xk_ref, s_ref,
               gated=True, gate_ref=gate_ref)


def _row_spec():
    return pl.BlockSpec((RB, F), lambda r: (r, 0))


def _col_spec():
    return pl.BlockSpec((RB, 1), lambda r: (r, 0))


def _full_spec(shape):
    return pl.BlockSpec(shape, lambda r: tuple(0 for _ in shape))


def _make_root(interpret=False):
    return pl.pallas_call(
        _root_body,
        grid=(NRB,),
        in_specs=[_row_spec(), _full_spec((F, F)), _full_spec((1, F))],
        out_specs=[_row_spec()],
        out_shape=[jax.ShapeDtypeStruct((NPAD, F), jnp.float32)],
        interpret=interpret,
    )


_root = _make_root()


def _conv_call(body, n_in, interpret=False):
    part_a = pl.BlockSpec((1, RB, F), lambda r: (0, r, 0))
    part_b = pl.BlockSpec((1, RB, F), lambda r: (1, r, 0))
    return pl.pallas_call(
        body,
        grid=(NRB,),
        in_specs=[_row_spec(), part_a, part_b]
        + ([_col_spec()] if n_in == 8 else [])
        + [_full_spec((F, F)), _full_spec((F, 1))],
        out_specs=[_row_spec(), _col_spec()],
        out_shape=[jax.ShapeDtypeStruct((NPAD, F), jnp.float32),
                   jax.ShapeDtypeStruct((NPAD, 1), jnp.float32)],
        interpret=interpret,
    )


def _chunk_range(bi_ref, cb_lo_ref, cb_hi_ref):
    blo = bi_ref[0, 0]
    bhi = bi_ref[RB - 1, 0]
    cb_lo = cb_lo_ref[...]
    cb_hi = cb_hi_ref[...]
    cidx = lax.broadcasted_iota(jnp.int32, (1, 128), 1)
    valid = cidx < NJC
    c0 = jnp.sum(jnp.where(valid & (cb_hi < blo), 1, 0))
    c1 = jnp.sum(jnp.where(valid & (cb_lo <= bhi), 1, 0))
    return c0, c1


def _rank1_body(si_ref, bi_ref, sj_ref, bj_ref, xk_ref, cb_lo_ref, cb_hi_ref,
                kept_ref, rank_ref, k1_ref, z_ref):
    r = pl.program_id(0)
    si = si_ref[...]
    bi = bi_ref[...]
    ii = r * RB + lax.broadcasted_iota(jnp.int32, (RB, 1), 0)
    c0, c1 = _chunk_range(bi_ref, cb_lo_ref, cb_hi_ref)

    def jstep(c, acc):
        rank, cnt = acc
        sj = sj_ref[0:1, pl.ds(c * JC, JC)]
        bj = bj_ref[0:1, pl.ds(c * JC, JC)]
        jj = c * JC + lax.broadcasted_iota(jnp.int32, (1, JC), 1)
        same = bi == bj
        less = (sj > si) | ((sj == si) & (jj < ii))
        rank = rank + jnp.sum(jnp.where(same & less, 1.0, 0.0), axis=1, keepdims=True)
        cnt = cnt + jnp.sum(jnp.where(same, 1.0, 0.0), axis=1, keepdims=True)
        return rank, cnt

    rank, cnt = lax.fori_loop(c0, c1, jstep,
                              (jnp.zeros((RB, 1), jnp.float32),
                               jnp.zeros((RB, 1), jnp.float32)))
    k1 = jnp.ceil(0.8 * cnt)
    kept = jnp.where(rank < k1, 1.0, 0.0)
    kept_ref[...] = kept
    rank_ref[...] = rank
    k1_ref[...] = k1
    z_ref[...] = xk_ref[...] * kept


def _make_rank1(interpret=False):
    return pl.pallas_call(
        _rank1_body,
        grid=(NRB,),
        in_specs=[_col_spec(), _col_spec(), _full_spec((1, NPAD)),
                  _full_spec((1, NPAD)), _row_spec(),
                  _full_spec((1, 128)), _full_spec((1, 128))],
        out_specs=[_col_spec(), _col_spec(), _col_spec(), _row_spec()],
        out_shape=[jax.ShapeDtypeStruct((NPAD, 1), jnp.float32),
                   jax.ShapeDtypeStruct((NPAD, 1), jnp.float32),
                   jax.ShapeDtypeStruct((NPAD, 1), jnp.float32),
                   jax.ShapeDtypeStruct((NPAD, F), jnp.float32)],
        interpret=interpret,
    )


_rank1 = _make_rank1()


def _rank2_pool_body(si_ref, ki_ref, ri_ref, bi_ref, k1_ref,
                     sj_ref, kj_ref, rj_ref, bj_ref, cb_lo_ref, cb_hi_ref,
                     z_ref, xk2_ref, br_ref,
                     l1m_ref, l1x_ref, l1b_ref, l2_ref, l2b_ref,
                     out_ref, sum1_ref, sum2_ref, max1_ref, max2_ref, cnt_ref):
    r = pl.program_id(0)
    NEG = jnp.float32(-3.0e38)

    @pl.when(r == 0)
    def _():
        sum1_ref[...] = jnp.zeros((NG, F), jnp.float32)
        sum2_ref[...] = jnp.zeros((NG, F), jnp.float32)
        max1_ref[...] = jnp.full((NG, F), NEG, jnp.float32)
        max2_ref[...] = jnp.full((NG, F), NEG, jnp.float32)
        cnt_ref[...] = jnp.zeros((NG, 128), jnp.float32)

    INF = jnp.float32(jnp.inf)
    kept1c = ki_ref[...]
    keyi = jnp.where(kept1c > 0, -si_ref[...], INF)
    r1i = ri_ref[...]
    bi = bi_ref[...]
    c0, c1 = _chunk_range(bi_ref, cb_lo_ref, cb_hi_ref)

    def jstep(c, rank):
        sj = sj_ref[0:1, pl.ds(c * JC, JC)]
        kj = kj_ref[0:1, pl.ds(c * JC, JC)]
        rj = rj_ref[0:1, pl.ds(c * JC, JC)]
        bj = bj_ref[0:1, pl.ds(c * JC, JC)]
        keyj = jnp.where(kj > 0, -sj, INF)
        same = bi == bj
        less = (keyj < keyi) | ((keyj == keyi) & (rj < r1i))
        return rank + jnp.sum(jnp.where(same & less, 1.0, 0.0), axis=1, keepdims=True)

    rank2 = lax.fori_loop(c0, c1, jstep, jnp.zeros((RB, 1), jnp.float32))
    k2 = jnp.ceil(0.8 * k1_ref[...])
    kept2c = kept1c * jnp.where(rank2 < k2, 1.0, 0.0)

    z = z_ref[...]
    z2 = xk2_ref[...] * kept2c
    br = br_ref[...]
    gcol = lax.broadcasted_iota(jnp.int32, (NG, RB), 0)
    samegf = jnp.where(br == gcol, 1.0, 0.0)
    sum1_ref[...] += jnp.dot(samegf, z, preferred_element_type=jnp.float32)
    sum2_ref[...] += jnp.dot(samegf, z2, preferred_element_type=jnp.float32)
    cnt_ref[...] += jnp.sum(samegf, axis=1, keepdims=True)

    glo = bi_ref[0, 0]
    ghi = jnp.minimum(bi_ref[RB - 1, 0], NG - 1)

    def gstep(g, carry):
        m1 = (bi == g) & (kept1c > 0)
        m2 = (bi == g) & (kept2c > 0)
        mx1 = jnp.max(jnp.where(m1, z, NEG), axis=0, keepdims=True)
        mx2 = jnp.max(jnp.where(m2, z2, NEG), axis=0, keepdims=True)
        max1_ref[pl.ds(g, 1), :] = jnp.maximum(max1_ref[pl.ds(g, 1), :], mx1)
        max2_ref[pl.ds(g, 1), :] = jnp.maximum(max2_ref[pl.ds(g, 1), :], mx2)
        return carry

    lax.fori_loop(glo, ghi + 1, gstep, 0)

    @pl.when(r == NRB - 1)
    def _():
        cntg = cnt_ref[:, 0:1]
        k1g = jnp.ceil(0.8 * cntg)
        k2g = jnp.ceil(0.8 * k1g)
        mean1 = sum1_ref[...] / jnp.maximum(k1g, 1.0)
        mean2 = sum2_ref[...] / jnp.maximum(k2g, 1.0)
        max1 = jnp.where(k1g > 0, max1_ref[...], 0.0)
        max2 = jnp.where(k2g > 0, max2_ref[...], 0.0)
        meanT = mean1 + mean2
        maxT = max1 + max2
        hid = jnp.dot(meanT, l1m_ref[...], preferred_element_type=jnp.float32)
        hid = hid + jnp.dot(maxT, l1x_ref[...], preferred_element_type=jnp.float32)
        hid = jnp.maximum(hid + l1b_ref[...], 0.0)
        out = jnp.dot(hid, l2_ref[...], preferred_element_type=jnp.float32)
        out_ref[...] = jnp.maximum(out + l2b_ref[...], 0.0)


def _make_rank2_pool(interpret=False):
    return pl.pallas_call(
        _rank2_pool_body,
        grid=(NRB,),
        in_specs=[_col_spec(), _col_spec(), _col_spec(), _col_spec(), _col_spec(),
                  _full_spec((1, NPAD)), _full_spec((1, NPAD)),
                  _full_spec((1, NPAD)), _full_spec((1, NPAD)),
                  _full_spec((1, 128)), _full_spec((1, 128)),
                  _row_spec(), _row_spec(),
                  pl.BlockSpec((1, RB), lambda r: (0, r)),
                  _full_spec((F, F // 4)), _full_spec((F, F // 4)),
                  _full_spec((1, F // 4)), _full_spec((F // 4, F)),
                  _full_spec((1, F))],
        out_specs=pl.BlockSpec((NG, F), lambda r: (0, 0)),
        out_shape=jax.ShapeDtypeStruct((NG, F), jnp.float32),
        scratch_shapes=[pltpu.VMEM((NG, F), jnp.float32),
                        pltpu.VMEM((NG, F), jnp.float32),
                        pltpu.VMEM((NG, F), jnp.float32),
                        pltpu.VMEM((NG, F), jnp.float32),
                        pltpu.VMEM((NG, 128), jnp.float32)],
        interpret=interpret,
    )


_rank2_pool = _make_rank2_pool()


def _run(ops, x, edge_index, batch, conv1_rel_w, conv1_rel_b, conv1_root_w,
         conv2_rel_w, conv2_rel_b, conv2_root_w, pool1_w, pool2_w,
         lin1_w, lin1_b, lin2_w, lin2_b):
    sc_scatter, root_op, conv1_op, conv2_op, rank1_op, rank2_pool_op = ops
    f32 = jnp.float32
    x_pad = jnp.zeros((NPAD, F), f32).at[:N].set(x)
    batch_pad = jnp.full((NPAD,), NG, jnp.int32).at[:N].set(batch)
    b_col = batch_pad[:, None]
    b_row = batch_pad[None, :]
    pad_cnt = EPAD - E
    pad_idx = N + (jnp.arange(pad_cnt, dtype=jnp.int32) % (NPAD - N))
    src_pad = jnp.concatenate([edge_index[0], pad_idx]).reshape(NW, NCHUNK, CH)
    dst_pad = jnp.concatenate([edge_index[1], pad_idx]).reshape(NW, NCHUNK, CH)
    zero_buf = jnp.zeros((NPAD, F), f32)

    w1r_t = conv1_rel_w.T
    w1q_t = conv1_root_w.T
    w2r_t = conv2_rel_w.T
    w2q_t = conv2_root_w.T
    b1 = conv1_rel_b[None, :]
    b2 = conv2_rel_b[None, :]
    pw1 = pool1_w[:, None]
    pw2 = pool2_w[:, None]
    l1m = lin1_w[:, :F].T
    l1x = lin1_w[:, F:].T
    l1b = lin1_b[None, :]
    l2_t = jnp.zeros((F // 4, F), f32).at[:, :10].set(lin2_w.T)
    l2b = jnp.zeros((1, F), f32).at[0, :10].set(lin2_b)
    cb_lo = jnp.zeros((1, 128), jnp.int32).at[0, :NJC].set(batch_pad[::JC])
    cb_hi = jnp.zeros((1, 128), jnp.int32).at[0, :NJC].set(batch_pad[JC - 1::JC])

    p = sc_scatter(x_pad, src_pad, dst_pad, zero_buf)
    xroot1, = root_op(x_pad, w1q_t, b1)
    xk1, s1 = conv1_op(xroot1, p, p, w1r_t, pw1)

    kept1, rank1, k1f, z = rank1_op(s1, b_col, s1.reshape(1, NPAD), b_row, xk1,
                                    cb_lo, cb_hi)

    q = sc_scatter(z, src_pad, dst_pad, zero_buf)
    xroot2, = root_op(xk1, w2q_t, b2)
    xk2, s2 = conv2_op(xroot2, q, q, kept1, w2r_t, pw2)

    out = rank2_pool_op(s2, kept1, rank1, b_col, k1f,
                        s2.reshape(1, NPAD), kept1.reshape(1, NPAD),
                        rank1.reshape(1, NPAD), b_row, cb_lo, cb_hi,
                        z, xk2, b_row, l1m, l1x, l1b, l2_t, l2b)
    return out[:, :10]


_DEFAULT_OPS = (_sc_scatter, _root, _conv_call(_conv1_body, 7),
                _conv_call(_conv2_body, 8), _rank1, _rank2_pool)


def kernel(x, edge_index, batch, conv1_rel_w, conv1_rel_b, conv1_root_w,
           conv2_rel_w, conv2_rel_b, conv2_root_w, pool1_w, pool2_w,
           lin1_w, lin1_b, lin2_w, lin2_b):
    return _run(_DEFAULT_OPS, x, edge_index, batch, conv1_rel_w, conv1_rel_b,
                conv1_root_w, conv2_rel_w, conv2_rel_b, conv2_root_w,
                pool1_w, pool2_w, lin1_w, lin1_b, lin2_w, lin2_b)


# speedup vs baseline: 1.0140x; 1.0140x over previous
import functools

import jax
import jax.numpy as jnp
from jax import lax
from jax.experimental import pallas as pl
from jax.experimental.pallas import tpu as pltpu
from jax.experimental.pallas import tpu_sc as plsc

N = 10000
NPAD = 10240
E = 320000
F = 128
NG = 64
RB = 512
NRB = NPAD // RB
JC = 512
NJC = NPAD // JC

NC = 2
NS = 16
NW = NC * NS
CH = 128
NCHUNK = 80
HC = NCHUNK // 2
NBUF = 2
EPW = NCHUNK * CH
EPAD = NW * EPW
ROWS_PER_SUB = NPAD // NS


def _sc_scatter_body(table_hbm, srcidx_hbm, dstidx_hbm, zero_hbm, out_hbm,
                     sidx_v, didx_v, rows0, rows1, acc_sh, gsem):
    rows = (rows0, rows1)
    cid = lax.axis_index("c")
    sid = lax.axis_index("s")
    w = cid * NS + sid
    r0 = sid * ROWS_PER_SUB
    pltpu.sync_copy(zero_hbm.at[pl.ds(r0, ROWS_PER_SUB)],
                    acc_sh.at[pl.ds(r0, ROWS_PER_SUB)])
    plsc.subcore_barrier()

    def _wait_gather(b):
        pltpu.make_async_copy(table_hbm.at[sidx_v.at[0]], rows[b],
                              gsem.at[b]).wait()

    for h in range(2):
        pltpu.sync_copy(srcidx_hbm.at[w, pl.ds(h * HC, HC)], sidx_v)
        pltpu.sync_copy(dstidx_hbm.at[w, pl.ds(h * HC, HC)], didx_v)
        pltpu.async_copy(table_hbm.at[sidx_v.at[0]], rows[0], gsem.at[0])

        def body(p, carry):
            j0 = 2 * p
            pltpu.async_copy(table_hbm.at[sidx_v.at[j0 + 1]], rows[1],
                             gsem.at[1])
            _wait_gather(0)
            pltpu.sync_copy(rows[0], acc_sh.at[didx_v.at[j0]], add=True)
            nxt = lax.rem(j0 + 2, HC)
            pltpu.async_copy(table_hbm.at[sidx_v.at[nxt]], rows[0],
                             gsem.at[0])
            _wait_gather(1)
            pltpu.sync_copy(rows[1], acc_sh.at[didx_v.at[j0 + 1]], add=True)
            return carry

        lax.fori_loop(0, HC // 2, body, 0, unroll=False)
        _wait_gather(0)
    plsc.subcore_barrier()
    pltpu.sync_copy(acc_sh.at[pl.ds(r0, ROWS_PER_SUB)],
                    out_hbm.at[cid, pl.ds(r0, ROWS_PER_SUB)])


def _make_sc_scatter(interpret=False):
    return pl.kernel(
        _sc_scatter_body,
        out_type=jax.ShapeDtypeStruct((NC, NPAD, F), jnp.float32),
        mesh=plsc.VectorSubcoreMesh(core_axis_name="c", subcore_axis_name="s",
                                    num_cores=NC, num_subcores=NS),
        scratch_types=[
            pltpu.VMEM((HC, CH), jnp.int32),
            pltpu.VMEM((HC, CH), jnp.int32),
            pltpu.VMEM((CH, F), jnp.float32),
            pltpu.VMEM((CH, F), jnp.float32),
            pltpu.VMEM_SHARED((NPAD, F), jnp.float32),
            pltpu.SemaphoreType.DMA((NBUF,)),
        ],
        interpret=interpret,
    )


_sc_scatter_cache = []


def _sc_scatter(*args):
    if not _sc_scatter_cache:
        _sc_scatter_cache.append(_make_sc_scatter())
    return _sc_scatter_cache[0](*args)


def _conv_body(x_ref, pa_ref, pb_ref, wr_ref, wq_ref, b_ref, pw_ref,
               xk_ref, s_ref, *, gated, gate_ref=None):
    agg = pa_ref[0] + pb_ref[0]
    if gated:
        agg = agg * gate_ref[...]
    h = jnp.dot(agg, wr_ref[...], preferred_element_type=jnp.float32)
    h = h + jnp.dot(x_ref[...], wq_ref[...], preferred_element_type=jnp.float32)
    h = jnp.maximum(h + b_ref[...], 0.0)
    w = pw_ref[...]
    nrm = jnp.sqrt(jnp.sum(w * w))
    s = jnp.tanh(jnp.dot(h, w, preferred_element_type=jnp.float32) / nrm)
    s_ref[...] = s
    xk_ref[...] = h * s


def _conv1_body(x_ref, pa_ref, pb_ref, wr_ref, wq_ref, b_ref, pw_ref, xk_ref, s_ref):
    _conv_body(x_ref, pa_ref, pb_ref, wr_ref, wq_ref, b_ref, pw_ref, xk_ref, s_ref,
               gated=False)


def _conv2_body(x_ref, pa_ref, pb_ref, gate_ref, wr_ref, wq_ref, b_ref, pw_ref,
                xk_ref, s_ref):
    _conv_body(x_ref, pa_ref, pb_ref, wr_ref, wq_ref, b_ref, pw_ref, xk_ref, s_ref,
               gated=True, gate_ref=gate_ref)


def _row_spec():
    return pl.BlockSpec((RB, F), lambda r: (r, 0))


def _col_spec():
    return pl.BlockSpec((RB, 1), lambda r: (r, 0))


def _full_spec(shape):
    return pl.BlockSpec(shape, lambda r: tuple(0 for _ in shape))


def _conv_call(body, n_in, interpret=False):
    part_a = pl.BlockSpec((1, RB, F), lambda r: (0, r, 0))
    part_b = pl.BlockSpec((1, RB, F), lambda r: (1, r, 0))
    return pl.pallas_call(
        body,
        grid=(NRB,),
        in_specs=[_row_spec(), part_a, part_b]
        + ([_col_spec()] if n_in == 8 else [])
        + [_full_spec((F, F)), _full_spec((F, F)), _full_spec((1, F)),
           _full_spec((F, 1))],
        out_specs=[_row_spec(), _col_spec()],
        out_shape=[jax.ShapeDtypeStruct((NPAD, F), jnp.float32),
                   jax.ShapeDtypeStruct((NPAD, 1), jnp.float32)],
        interpret=interpret,
    )


def _chunk_range(bi_ref, cb_lo_ref, cb_hi_ref):
    blo = bi_ref[0, 0]
    bhi = bi_ref[RB - 1, 0]
    cb_lo = cb_lo_ref[...]
    cb_hi = cb_hi_ref[...]
    cidx = lax.broadcasted_iota(jnp.int32, (1, 128), 1)
    valid = cidx < NJC
    c0 = jnp.sum(jnp.where(valid & (cb_hi < blo), 1, 0))
    c1 = jnp.sum(jnp.where(valid & (cb_lo <= bhi), 1, 0))
    return c0, c1


def _rank1_body(si_ref, bi_ref, sj_ref, bj_ref, xk_ref, cb_lo_ref, cb_hi_ref,
                kept_ref, rank_ref, k1_ref, z_ref):
    r = pl.program_id(0)
    si = si_ref[...]
    bi = bi_ref[...]
    ii = r * RB + lax.broadcasted_iota(jnp.int32, (RB, 1), 0)
    c0, c1 = _chunk_range(bi_ref, cb_lo_ref, cb_hi_ref)

    def jstep(c, acc):
        rank, cnt = acc
        sj = sj_ref[0:1, pl.ds(c * JC, JC)]
        bj = bj_ref[0:1, pl.ds(c * JC, JC)]
        jj = c * JC + lax.broadcasted_iota(jnp.int32, (1, JC), 1)
        same = bi == bj
        less = (sj > si) | ((sj == si) & (jj < ii))
        rank = rank + jnp.sum(jnp.where(same & less, 1.0, 0.0), axis=1, keepdims=True)
        cnt = cnt + jnp.sum(jnp.where(same, 1.0, 0.0), axis=1, keepdims=True)
        return rank, cnt

    rank, cnt = lax.fori_loop(c0, c1, jstep,
                              (jnp.zeros((RB, 1), jnp.float32),
                               jnp.zeros((RB, 1), jnp.float32)))
    k1 = jnp.ceil(0.8 * cnt)
    kept = jnp.where(rank < k1, 1.0, 0.0)
    kept_ref[...] = kept
    rank_ref[...] = rank
    k1_ref[...] = k1
    z_ref[...] = xk_ref[...] * kept


def _make_rank1(interpret=False):
    return pl.pallas_call(
        _rank1_body,
        grid=(NRB,),
        in_specs=[_col_spec(), _col_spec(), _full_spec((1, NPAD)),
                  _full_spec((1, NPAD)), _row_spec(),
                  _full_spec((1, 128)), _full_spec((1, 128))],
        out_specs=[_col_spec(), _col_spec(), _col_spec(), _row_spec()],
        out_shape=[jax.ShapeDtypeStruct((NPAD, 1), jnp.float32),
                   jax.ShapeDtypeStruct((NPAD, 1), jnp.float32),
                   jax.ShapeDtypeStruct((NPAD, 1), jnp.float32),
                   jax.ShapeDtypeStruct((NPAD, F), jnp.float32)],
        interpret=interpret,
    )


_rank1 = _make_rank1()


def _rank2_pool_body(si_ref, ki_ref, ri_ref, bi_ref, k1_ref,
                     sj_ref, kj_ref, rj_ref, bj_ref, cb_lo_ref, cb_hi_ref,
                     z_ref, xk2_ref, br_ref,
                     l1m_ref, l1x_ref, l1b_ref, l2_ref, l2b_ref,
                     out_ref, sum1_ref, sum2_ref, max1_ref, max2_ref, cnt_ref):
    r = pl.program_id(0)
    NEG = jnp.float32(-3.0e38)

    @pl.when(r == 0)
    def _():
        sum1_ref[...] = jnp.zeros((NG, F), jnp.float32)
        sum2_ref[...] = jnp.zeros((NG, F), jnp.float32)
        max1_ref[...] = jnp.full((NG, F), NEG, jnp.float32)
        max2_ref[...] = jnp.full((NG, F), NEG, jnp.float32)
        cnt_ref[...] = jnp.zeros((NG, 128), jnp.float32)

    INF = jnp.float32(jnp.inf)
    kept1c = ki_ref[...]
    keyi = jnp.where(kept1c > 0, -si_ref[...], INF)
    r1i = ri_ref[...]
    bi = bi_ref[...]
    c0, c1 = _chunk_range(bi_ref, cb_lo_ref, cb_hi_ref)

    def jstep(c, rank):
        sj = sj_ref[0:1, pl.ds(c * JC, JC)]
        kj = kj_ref[0:1, pl.ds(c * JC, JC)]
        rj = rj_ref[0:1, pl.ds(c * JC, JC)]
        bj = bj_ref[0:1, pl.ds(c * JC, JC)]
        keyj = jnp.where(kj > 0, -sj, INF)
        same = bi == bj
        less = (keyj < keyi) | ((keyj == keyi) & (rj < r1i))
        return rank + jnp.sum(jnp.where(same & less, 1.0, 0.0), axis=1, keepdims=True)

    rank2 = lax.fori_loop(c0, c1, jstep, jnp.zeros((RB, 1), jnp.float32))
    k2 = jnp.ceil(0.8 * k1_ref[...])
    kept2c = kept1c * jnp.where(rank2 < k2, 1.0, 0.0)

    z = z_ref[...]
    z2 = xk2_ref[...] * kept2c
    br = br_ref[...]
    gcol = lax.broadcasted_iota(jnp.int32, (NG, RB), 0)
    samegf = jnp.where(br == gcol, 1.0, 0.0)
    sum1_ref[...] += jnp.dot(samegf, z, preferred_element_type=jnp.float32)
    sum2_ref[...] += jnp.dot(samegf, z2, preferred_element_type=jnp.float32)
    cnt_ref[...] += jnp.sum(samegf, axis=1, keepdims=True)

    glo = bi_ref[0, 0]
    ghi = jnp.minimum(bi_ref[RB - 1, 0], NG - 1)

    def gstep(g, carry):
        m1 = (bi == g) & (kept1c > 0)
        m2 = (bi == g) & (kept2c > 0)
        mx1 = jnp.max(jnp.where(m1, z, NEG), axis=0, keepdims=True)
        mx2 = jnp.max(jnp.where(m2, z2, NEG), axis=0, keepdims=True)
        max1_ref[pl.ds(g, 1), :] = jnp.maximum(max1_ref[pl.ds(g, 1), :], mx1)
        max2_ref[pl.ds(g, 1), :] = jnp.maximum(max2_ref[pl.ds(g, 1), :], mx2)
        return carry

    lax.fori_loop(glo, ghi + 1, gstep, 0)

    @pl.when(r == NRB - 1)
    def _():
        cntg = cnt_ref[:, 0:1]
        k1g = jnp.ceil(0.8 * cntg)
        k2g = jnp.ceil(0.8 * k1g)
        mean1 = sum1_ref[...] / jnp.maximum(k1g, 1.0)
        mean2 = sum2_ref[...] / jnp.maximum(k2g, 1.0)
        max1 = jnp.where(k1g > 0, max1_ref[...], 0.0)
        max2 = jnp.where(k2g > 0, max2_ref[...], 0.0)
        meanT = mean1 + mean2
        maxT = max1 + max2
        hid = jnp.dot(meanT, l1m_ref[...], preferred_element_type=jnp.float32)
        hid = hid + jnp.dot(maxT, l1x_ref[...], preferred_element_type=jnp.float32)
        hid = jnp.maximum(hid + l1b_ref[...], 0.0)
        out = jnp.dot(hid, l2_ref[...], preferred_element_type=jnp.float32)
        out_ref[...] = jnp.maximum(out + l2b_ref[...], 0.0)


def _make_rank2_pool(interpret=False):
    return pl.pallas_call(
        _rank2_pool_body,
        grid=(NRB,),
        in_specs=[_col_spec(), _col_spec(), _col_spec(), _col_spec(), _col_spec(),
                  _full_spec((1, NPAD)), _full_spec((1, NPAD)),
                  _full_spec((1, NPAD)), _full_spec((1, NPAD)),
                  _full_spec((1, 128)), _full_spec((1, 128)),
                  _row_spec(), _row_spec(),
                  pl.BlockSpec((1, RB), lambda r: (0, r)),
                  _full_spec((F, F // 4)), _full_spec((F, F // 4)),
                  _full_spec((1, F // 4)), _full_spec((F // 4, F)),
                  _full_spec((1, F))],
        out_specs=pl.BlockSpec((NG, F), lambda r: (0, 0)),
        out_shape=jax.ShapeDtypeStruct((NG, F), jnp.float32),
        scratch_shapes=[pltpu.VMEM((NG, F), jnp.float32),
                        pltpu.VMEM((NG, F), jnp.float32),
                        pltpu.VMEM((NG, F), jnp.float32),
                        pltpu.VMEM((NG, F), jnp.float32),
                        pltpu.VMEM((NG, 128), jnp.float32)],
        interpret=interpret,
    )


_rank2_pool = _make_rank2_pool()


def _run(ops, x, edge_index, batch, conv1_rel_w, conv1_rel_b, conv1_root_w,
         conv2_rel_w, conv2_rel_b, conv2_root_w, pool1_w, pool2_w,
         lin1_w, lin1_b, lin2_w, lin2_b):
    sc_scatter, conv1_op, conv2_op, rank1_op, rank2_pool_op = ops
    f32 = jnp.float32
    x_pad = jnp.zeros((NPAD, F), f32).at[:N].set(x)
    batch_pad = jnp.full((NPAD,), NG, jnp.int32).at[:N].set(batch)
    b_col = batch_pad[:, None]
    b_row = batch_pad[None, :]
    pad_cnt = EPAD - E
    pad_idx = N + (jnp.arange(pad_cnt, dtype=jnp.int32) % (NPAD - N))
    src_pad = jnp.concatenate([edge_index[0], pad_idx]).reshape(NW, NCHUNK, CH)
    dst_pad = jnp.concatenate([edge_index[1], pad_idx]).reshape(NW, NCHUNK, CH)
    zero_buf = jnp.zeros((NPAD, F), f32)

    w1r_t = conv1_rel_w.T
    w1q_t = conv1_root_w.T
    w2r_t = conv2_rel_w.T
    w2q_t = conv2_root_w.T
    b1 = conv1_rel_b[None, :]
    b2 = conv2_rel_b[None, :]
    pw1 = pool1_w[:, None]
    pw2 = pool2_w[:, None]
    l1m = lin1_w[:, :F].T
    l1x = lin1_w[:, F:].T
    l1b = lin1_b[None, :]
    l2_t = jnp.zeros((F // 4, F), f32).at[:, :10].set(lin2_w.T)
    l2b = jnp.zeros((1, F), f32).at[0, :10].set(lin2_b)
    cb_lo = jnp.zeros((1, 128), jnp.int32).at[0, :NJC].set(batch_pad[::JC])
    cb_hi = jnp.zeros((1, 128), jnp.int32).at[0, :NJC].set(batch_pad[JC - 1::JC])

    p = sc_scatter(x_pad, src_pad, dst_pad, zero_buf)
    xk1, s1 = conv1_op(x_pad, p, p, w1r_t, w1q_t, b1, pw1)

    kept1, rank1, k1f, z = rank1_op(s1, b_col, s1.reshape(1, NPAD), b_row, xk1,
                                    cb_lo, cb_hi)

    q = sc_scatter(z, src_pad, dst_pad, zero_buf)
    xk2, s2 = conv2_op(xk1, q, q, kept1, w2r_t, w2q_t, b2, pw2)

    out = rank2_pool_op(s2, kept1, rank1, b_col, k1f,
                        s2.reshape(1, NPAD), kept1.reshape(1, NPAD),
                        rank1.reshape(1, NPAD), b_row, cb_lo, cb_hi,
                        z, xk2, b_row, l1m, l1x, l1b, l2_t, l2b)
    return out[:, :10]


_DEFAULT_OPS = (_sc_scatter, _conv_call(_conv1_body, 7), _conv_call(_conv2_body, 8),
                _rank1, _rank2_pool)


def kernel(x, edge_index, batch, conv1_rel_w, conv1_rel_b, conv1_root_w,
           conv2_rel_w, conv2_rel_b, conv2_root_w, pool1_w, pool2_w,
           lin1_w, lin1_b, lin2_w, lin2_b):
    return _run(_DEFAULT_OPS, x, edge_index, batch, conv1_rel_w, conv1_rel_b,
                conv1_root_w, conv2_rel_w, conv2_rel_b, conv2_root_w,
                pool1_w, pool2_w, lin1_w, lin1_b, lin2_w, lin2_b)
